# baseline (speedup 1.0000x reference)
import functools

import jax
import jax.numpy as jnp
from jax import lax
from jax.experimental import pallas as pl
from jax.experimental.pallas import tpu as pltpu
from jax.experimental.pallas import tpu_sc as plsc

_B = 16384
_D = 64
_NC = 2
_NS = 16
_NW = _NC * _NS
_BPW = _B // _NW
_CH = 128
_L = 16
_VP = 500000


def _tt_kernel(user_idx, item_idx, utp, itp, out_hbm,
               uidx_v, iidx_v, upair_v, ipair_v,
               gu_v, gi_v, out_v, sem_u, sem_i):
    wid = lax.axis_index("s") * _NC + lax.axis_index("c")
    lane_iota = lax.iota(jnp.int32, _L)

    def chunk(h, carry):
        base = wid * _BPW + h * _CH
        pltpu.sync_copy(user_idx.at[pl.ds(base, _CH)], uidx_v)
        pltpu.sync_copy(item_idx.at[pl.ds(base, _CH)], iidx_v)

        def pairify(g, carry):
            s = pl.ds(g * _L, _L)
            upair_v[s] = uidx_v[s] >> 1
            ipair_v[s] = iidx_v[s] >> 1
            return carry

        lax.fori_loop(0, _CH // _L, pairify, 0)

        cu = pltpu.async_copy(utp.at[upair_v], gu_v, sem_u)
        ci = pltpu.async_copy(itp.at[ipair_v], gi_v, sem_i)
        cu.wait()
        ci.wait()

        def blk(g, carry):
            r0 = g * _L
            acc_ll = jnp.zeros((_L,), jnp.float32)
            acc_lh = jnp.zeros((_L,), jnp.float32)
            acc_hl = jnp.zeros((_L,), jnp.float32)
            acc_hh = jnp.zeros((_L,), jnp.float32)
            for j in range(_L):
                row = r0 + j
                u = [gu_v[row, pl.ds(c * _L, _L)] for c in range(8)]
                v = [gi_v[row, pl.ds(c * _L, _L)] for c in range(8)]
                ll = u[0] * v[0] + u[1] * v[1] + u[2] * v[2] + u[3] * v[3]
                lh = u[0] * v[4] + u[1] * v[5] + u[2] * v[6] + u[3] * v[7]
                hl = u[4] * v[0] + u[5] * v[1] + u[6] * v[2] + u[7] * v[3]
                hh = u[4] * v[4] + u[5] * v[5] + u[6] * v[6] + u[7] * v[7]
                sel = lane_iota == j
                acc_ll = jnp.where(sel, jnp.sum(ll), acc_ll)
                acc_lh = jnp.where(sel, jnp.sum(lh), acc_lh)
                acc_hl = jnp.where(sel, jnp.sum(hl), acc_hl)
                acc_hh = jnp.where(sel, jnp.sum(hh), acc_hh)
            s = pl.ds(r0, _L)
            pu = (uidx_v[s] & 1) == 1
            pv = (iidx_v[s] & 1) == 1
            out_v[pl.ds(h * _CH + r0, _L)] = jnp.where(
                pu, jnp.where(pv, acc_hh, acc_hl),
                jnp.where(pv, acc_lh, acc_ll))
            return carry

        lax.fori_loop(0, _CH // _L, blk, 0)
        return carry

    lax.fori_loop(0, _BPW // _CH, chunk, 0)

    pltpu.sync_copy(out_v, out_hbm.at[pl.ds(wid * _BPW, _BPW)])


@jax.jit
def kernel(user_idx, item_idx, user_table, item_table):
    mesh = plsc.VectorSubcoreMesh(core_axis_name="c", subcore_axis_name="s")
    f = functools.partial(
        pl.kernel,
        out_type=jax.ShapeDtypeStruct((_B,), jnp.float32),
        mesh=mesh,
        compiler_params=pltpu.CompilerParams(needs_layout_passes=False),
        scratch_types=[
            pltpu.VMEM((_CH,), jnp.int32),
            pltpu.VMEM((_CH,), jnp.int32),
            pltpu.VMEM((_CH,), jnp.int32),
            pltpu.VMEM((_CH,), jnp.int32),
            pltpu.VMEM((_CH, 2 * _D), jnp.float32),
            pltpu.VMEM((_CH, 2 * _D), jnp.float32),
            pltpu.VMEM((_BPW,), jnp.float32),
            pltpu.SemaphoreType.DMA,
            pltpu.SemaphoreType.DMA,
        ],
    )(_tt_kernel)
    return f(user_idx.astype(jnp.int32), item_idx.astype(jnp.int32),
             user_table.reshape(_VP, 2 * _D), item_table.reshape(_VP, 2 * _D))


# speedup vs baseline: 1.2223x; 1.2223x over previous
import functools

import jax
import jax.numpy as jnp
from jax import lax
from jax.experimental import pallas as pl
from jax.experimental.pallas import tpu as pltpu
from jax.experimental.pallas import tpu_sc as plsc

_B = 16384
_D = 64
_NC = 2
_NS = 16
_NW = _NC * _NS
_BPW = _B // _NW
_CH = 256
_L = 16


def _tt_kernel(user_idx, item_idx, tab, out_hbm,
               uidx_v, iidx_v, gu_v, gi_v, out_v, sem_u, sem_i):
    wid = lax.axis_index("s") * _NC + lax.axis_index("c")
    lane_iota = lax.iota(jnp.int32, _L)

    def chunk(h, carry):
        base = wid * _BPW + h * _CH
        pltpu.sync_copy(user_idx.at[pl.ds(base, _CH)], uidx_v)
        pltpu.sync_copy(item_idx.at[pl.ds(base, _CH)], iidx_v)

        cu = pltpu.async_copy(tab.at[uidx_v], gu_v, sem_u)
        ci = pltpu.async_copy(tab.at[iidx_v], gi_v, sem_i)
        cu.wait()
        ci.wait()

        def blk(g, carry):
            r0 = g * _L
            acc = jnp.zeros((_L,), jnp.float32)
            for j in range(_L):
                row = r0 + j
                pu = (gu_v[row, pl.ds(0, _L)] * gi_v[row, pl.ds(_D, _L)]
                      + gu_v[row, pl.ds(_L, _L)]
                      * gi_v[row, pl.ds(_D + _L, _L)]
                      + gu_v[row, pl.ds(2 * _L, _L)]
                      * gi_v[row, pl.ds(_D + 2 * _L, _L)]
                      + gu_v[row, pl.ds(3 * _L, _L)]
                      * gi_v[row, pl.ds(_D + 3 * _L, _L)])
                acc = jnp.where(lane_iota == j, jnp.sum(pu), acc)
            out_v[pl.ds(h * _CH + r0, _L)] = acc
            return carry

        lax.fori_loop(0, _CH // _L, blk, 0)
        return carry

    lax.fori_loop(0, _BPW // _CH, chunk, 0)

    pltpu.sync_copy(out_v, out_hbm.at[pl.ds(wid * _BPW, _BPW)])


@jax.jit
def kernel(user_idx, item_idx, user_table, item_table):
    mesh = plsc.VectorSubcoreMesh(core_axis_name="c", subcore_axis_name="s")
    f = functools.partial(
        pl.kernel,
        out_type=jax.ShapeDtypeStruct((_B,), jnp.float32),
        mesh=mesh,
        compiler_params=pltpu.CompilerParams(needs_layout_passes=False),
        scratch_types=[
            pltpu.VMEM((_CH,), jnp.int32),
            pltpu.VMEM((_CH,), jnp.int32),
            pltpu.VMEM((_CH, 2 * _D), jnp.float32),
            pltpu.VMEM((_CH, 2 * _D), jnp.float32),
            pltpu.VMEM((_BPW,), jnp.float32),
            pltpu.SemaphoreType.DMA,
            pltpu.SemaphoreType.DMA,
        ],
    )(_tt_kernel)
    tab = jnp.concatenate([user_table, item_table], axis=1)
    return f(user_idx.astype(jnp.int32), item_idx.astype(jnp.int32), tab)
